# baseline (speedup 1.0000x reference)
import functools

import jax
import jax.numpy as jnp
from jax.experimental import pallas as pl
from jax.experimental.pallas import tpu as pltpu

_N, _D = 8, 2048
_NEG_INF = -1e30


def _fused_body(x_ref, w_ref, b_ref, g_ref, act_ref, logp_ref):
    w = w_ref[0]
    logits = jax.lax.dot_general(
        x_ref[0], w,
        dimension_numbers=(((1,), (1,)), ((), ())),
        preferred_element_type=jnp.float32,
        precision=jax.lax.Precision.HIGHEST,
    ) + b_ref[0]

    perturbed = logits + g_ref[0]
    action = jnp.argmax(perturbed, axis=1)[0]

    m = jnp.max(logits)
    lse = m + jnp.log(jnp.sum(jnp.exp(logits - m)))
    kidx = jax.lax.broadcasted_iota(jnp.int32, (1, _D), 1)
    logit_at = jnp.max(jnp.where(kidx == action, logits, _NEG_INF))
    logp = logit_at - lse

    act_ref[0] = jnp.full((1, 128), action, jnp.int32)
    logp_ref[0] = jnp.full((1, 128), logp, jnp.float32)


@jax.jit
def kernel(x, W, b):
    u = jax.random.uniform(jax.random.key(42), (_N, _D), dtype=jnp.float32)
    g = -jnp.log(-jnp.log(u + 1e-20) + 1e-20)

    acts, logps = pl.pallas_call(
        _fused_body,
        grid=(_N,),
        in_specs=[
            pl.BlockSpec((1, 1, _D), lambda n: (n, 0, 0)),
            pl.BlockSpec((1, _D, _D), lambda n: (n, 0, 0)),
            pl.BlockSpec((1, 1, _D), lambda n: (n, 0, 0)),
            pl.BlockSpec((1, 1, _D), lambda n: (n, 0, 0)),
        ],
        out_specs=[
            pl.BlockSpec((1, 1, 128), lambda n: (n, 0, 0)),
            pl.BlockSpec((1, 1, 128), lambda n: (n, 0, 0)),
        ],
        out_shape=[
            jax.ShapeDtypeStruct((_N, 1, 128), jnp.int32),
            jax.ShapeDtypeStruct((_N, 1, 128), jnp.float32),
        ],
    )(x[:, None, :], W, b[:, None, :], g[:, None, :])

    actions = acts[:, 0, :1].astype(jnp.int64)
    return actions, logps[:, 0, :1]


# speedup vs baseline: 2.3264x; 2.3264x over previous
import functools

import jax
import jax.numpy as jnp
from jax.experimental import pallas as pl
from jax.experimental.pallas import tpu as pltpu

_N, _D = 8, 2048
_NEG_INF = -1e30


def _fused_body(x_ref, w_ref, b_ref, g_ref, act_ref, logp_ref):
    w = w_ref[0]
    logits = jax.lax.dot_general(
        x_ref[0], w,
        dimension_numbers=(((1,), (1,)), ((), ())),
        preferred_element_type=jnp.float32,
        precision=jax.lax.Precision.DEFAULT,
    ) + b_ref[0]

    perturbed = logits + g_ref[0]
    action = jnp.argmax(perturbed, axis=1)[0]

    m = jnp.max(logits)
    lse = m + jnp.log(jnp.sum(jnp.exp(logits - m)))
    kidx = jax.lax.broadcasted_iota(jnp.int32, (1, _D), 1)
    logit_at = jnp.max(jnp.where(kidx == action, logits, _NEG_INF))
    logp = logit_at - lse

    act_ref[0] = jnp.full((1, 128), action, jnp.int32)
    logp_ref[0] = jnp.full((1, 128), logp, jnp.float32)


@jax.jit
def kernel(x, W, b):
    u = jax.random.uniform(jax.random.key(42), (_N, _D), dtype=jnp.float32)
    g = -jnp.log(-jnp.log(u + 1e-20) + 1e-20)

    acts, logps = pl.pallas_call(
        _fused_body,
        grid=(_N,),
        in_specs=[
            pl.BlockSpec((1, 1, _D), lambda n: (n, 0, 0)),
            pl.BlockSpec((1, _D, _D), lambda n: (n, 0, 0)),
            pl.BlockSpec((1, 1, _D), lambda n: (n, 0, 0)),
            pl.BlockSpec((1, 1, _D), lambda n: (n, 0, 0)),
        ],
        out_specs=[
            pl.BlockSpec((1, 1, 128), lambda n: (n, 0, 0)),
            pl.BlockSpec((1, 1, 128), lambda n: (n, 0, 0)),
        ],
        out_shape=[
            jax.ShapeDtypeStruct((_N, 1, 128), jnp.int32),
            jax.ShapeDtypeStruct((_N, 1, 128), jnp.float32),
        ],
    )(x[:, None, :], W, b[:, None, :], g[:, None, :])

    actions = acts[:, 0, :1].astype(jnp.int64)
    return actions, logps[:, 0, :1]
